# Initial kernel scaffold; baseline (speedup 1.0000x reference)
import jax
import jax.numpy as jnp
from jax.experimental import pallas as pl


def kernel(mem, val, idx):
    raise NotImplementedError("write your pallas kernel here")


# speedup vs baseline: 1.7383x; 1.7383x over previous
import functools

import jax
import jax.numpy as jnp
from jax import lax
from jax.experimental import pallas as pl
from jax.experimental.pallas import tpu as pltpu
from jax.experimental.pallas import tpu_sc as plsc

CAP = 1000000
DIM = 64
B = 16384

_info = plsc.get_sparse_core_info()
NC, NS, L = _info.num_cores, _info.num_subcores, _info.num_lanes
NW = NC * NS
BPW = B // NW
NCHUNK = 4
CROWS = BPW // NCHUNK
TRASH = B


def _body(mem_h, val_h, idx_h, out_h, idx_v,
          adj0, adj1, adj2, adj3,
          aval0, aval1, aval2, aval3,
          dmem0, dmem1, dmem2, dmem3,
          dval0, dval1, dval2, dval3,
          rows_v, fix_v, sg1, sg2, ss1, ss2):
    adjb = (adj0, adj1, adj2, adj3)
    avalb = (aval0, aval1, aval2, aval3)
    dmemb = (dmem0, dmem1, dmem2, dmem3)
    dvalb = (dval0, dval1, dval2, dval3)

    wid = lax.axis_index("s") * NC + lax.axis_index("c")
    base = wid * BPW
    pltpu.sync_copy(idx_h.at[pl.ds(base, BPW)], idx_v)

    iota = lax.iota(jnp.int32, L)
    for c in range(BPW // L):
        j, o = divmod(c * L, CROWS)
        iv = idx_v[pl.ds(c * L, L)]
        adj = iv + B
        adj = jnp.where(adj >= CAP, adj - CAP, adj)
        m = adj < B
        dst = base + c * L + iota
        adjb[j][pl.ds(o, L)] = adj
        avalb[j][pl.ds(o, L)] = jnp.where(m, adj, 0)
        dmemb[j][pl.ds(o, L)] = jnp.where(m, TRASH, dst)
        dvalb[j][pl.ds(o, L)] = jnp.where(m, dst, TRASH)

    g1 = [pltpu.async_copy(mem_h.at[adjb[j]],
                           rows_v.at[pl.ds(j * CROWS, CROWS)], sg1)
          for j in range(NCHUNK)]
    g2 = [pltpu.async_copy(val_h.at[avalb[j]],
                           fix_v.at[pl.ds(j * CROWS, CROWS)], sg2)
          for j in range(NCHUNK)]
    for d in g1:
        d.wait()
    s1 = [pltpu.async_copy(rows_v.at[pl.ds(j * CROWS, CROWS)],
                           out_h.at[dmemb[j]], ss1)
          for j in range(NCHUNK)]
    for d in g2:
        d.wait()
    s2 = [pltpu.async_copy(fix_v.at[pl.ds(j * CROWS, CROWS)],
                           out_h.at[dvalb[j]], ss2)
          for j in range(NCHUNK)]
    for d in s1:
        d.wait()
    for d in s2:
        d.wait()


@jax.jit
def kernel(mem, val, idx):
    scratch = ([pltpu.VMEM((BPW,), jnp.int32)]
               + [pltpu.VMEM((CROWS,), jnp.int32) for _ in range(4 * NCHUNK)]
               + [pltpu.VMEM((BPW, DIM), jnp.float32) for _ in range(2)]
               + [pltpu.SemaphoreType.DMA for _ in range(4)])
    out = pl.kernel(
        _body,
        out_type=jax.ShapeDtypeStruct((B + 8, DIM), jnp.float32),
        scratch_types=scratch,
        mesh=plsc.VectorSubcoreMesh(core_axis_name="c", subcore_axis_name="s"),
        compiler_params=pltpu.CompilerParams(use_tc_tiling_on_sc=False),
    )(mem, val, idx)
    return out[:B]


# speedup vs baseline: 2.2615x; 1.3010x over previous
import jax
import jax.numpy as jnp
from jax import lax
from jax.experimental import pallas as pl
from jax.experimental.pallas import tpu as pltpu
from jax.experimental.pallas import tpu_sc as plsc

CAP = 1000000
DIM = 64
B = 16384

_info = plsc.get_sparse_core_info()
NC, NS, L = _info.num_cores, _info.num_subcores, _info.num_lanes
NW = NC * NS
BPW = B // NW
NCHUNK = 4
CROWS = BPW // NCHUNK


def _body(mem_h, val_h, idx_h, out_h, idx_v, adj_v,
          adj0, adj1, adj2, adj3,
          aval0, aval1, aval2, aval3,
          rows_v, fix_v, sg1, sg2):
    adjb = (adj0, adj1, adj2, adj3)
    avalb = (aval0, aval1, aval2, aval3)

    wid = lax.axis_index("s") * NC + lax.axis_index("c")
    base = wid * BPW
    pltpu.sync_copy(idx_h.at[pl.ds(base, BPW)], idx_v)

    for c in range(BPW // L):
        j, o = divmod(c * L, CROWS)
        iv = idx_v[pl.ds(c * L, L)]
        adj = iv + B
        adj = jnp.where(adj >= CAP, adj - CAP, adj)
        m = adj < B
        adjb[j][pl.ds(o, L)] = adj
        adj_v[pl.ds(c * L, L)] = adj
        avalb[j][pl.ds(o, L)] = jnp.where(m, adj, 0)

    g1 = [pltpu.async_copy(mem_h.at[adjb[j]],
                           rows_v.at[pl.ds(j * CROWS, CROWS)], sg1)
          for j in range(NCHUNK)]
    g2 = [pltpu.async_copy(val_h.at[avalb[j]],
                           fix_v.at[pl.ds(j * CROWS, CROWS)], sg2)
          for j in range(NCHUNK)]
    for d in g1:
        d.wait()
    for d in g2:
        d.wait()

    iota = lax.iota(jnp.int32, L)

    def patch(c, carry):
        adj = adj_v[pl.ds(c * L, L)]
        m = adj < B
        lid = c * L + iota
        for d in range(DIM):
            col = jnp.full((L,), d, jnp.int32)
            x = plsc.load_gather(fix_v, [lid, col], mask=m)
            plsc.store_scatter(rows_v, [lid, col], x, mask=m)
        return carry

    lax.fori_loop(0, BPW // L, patch, 0)

    pltpu.sync_copy(rows_v, out_h.at[pl.ds(base, BPW)])


@jax.jit
def kernel(mem, val, idx):
    scratch = ([pltpu.VMEM((BPW,), jnp.int32) for _ in range(2)]
               + [pltpu.VMEM((CROWS,), jnp.int32) for _ in range(2 * NCHUNK)]
               + [pltpu.VMEM((BPW, DIM), jnp.float32) for _ in range(2)]
               + [pltpu.SemaphoreType.DMA for _ in range(2)])
    return pl.kernel(
        _body,
        out_type=jax.ShapeDtypeStruct((B, DIM), jnp.float32),
        scratch_types=scratch,
        mesh=plsc.VectorSubcoreMesh(core_axis_name="c", subcore_axis_name="s"),
        compiler_params=pltpu.CompilerParams(use_tc_tiling_on_sc=False,
                                             needs_layout_passes=False),
    )(mem, val, idx)


# speedup vs baseline: 2.2637x; 1.0009x over previous
import jax
import jax.numpy as jnp
from jax import lax
from jax.experimental import pallas as pl
from jax.experimental.pallas import tpu as pltpu
from jax.experimental.pallas import tpu_sc as plsc

CAP = 1000000
DIM = 64
B = 16384

_info = plsc.get_sparse_core_info()
NC, NS, L = _info.num_cores, _info.num_subcores, _info.num_lanes
NW = NC * NS
BPW = B // NW
NCHUNK = 4
CROWS = BPW // NCHUNK


def _body(mem_h, val_h, idx_h, out_h, idx_v, adj_v,
          adj0, adj1, adj2, adj3,
          aval0, aval1, aval2, aval3,
          rows_v, fix_v, sg1, sg2, sw):
    adjb = (adj0, adj1, adj2, adj3)
    avalb = (aval0, aval1, aval2, aval3)

    wid = lax.axis_index("s") * NC + lax.axis_index("c")
    base = wid * BPW
    pltpu.sync_copy(idx_h.at[pl.ds(base, BPW)], idx_v)

    for c in range(BPW // L):
        j, o = divmod(c * L, CROWS)
        iv = idx_v[pl.ds(c * L, L)]
        adj = iv + B
        adj = jnp.where(adj >= CAP, adj - CAP, adj)
        m = adj < B
        adjb[j][pl.ds(o, L)] = adj
        adj_v[pl.ds(c * L, L)] = adj
        avalb[j][pl.ds(o, L)] = jnp.where(m, adj, 0)

    g1, g2 = [], []
    for j in range(NCHUNK):
        g1.append(pltpu.async_copy(mem_h.at[adjb[j]],
                                   rows_v.at[pl.ds(j * CROWS, CROWS)], sg1))
        g2.append(pltpu.async_copy(val_h.at[avalb[j]],
                                   fix_v.at[pl.ds(j * CROWS, CROWS)], sg2))

    iota = lax.iota(jnp.int32, L)
    ws = []
    for j in range(NCHUNK):
        g1[j].wait()
        g2[j].wait()

        def patch(c, carry):
            adj = adj_v[pl.ds(c * L, L)]
            m = adj < B
            lid = c * L + iota
            for d in range(DIM):
                col = jnp.full((L,), d, jnp.int32)
                x = plsc.load_gather(fix_v, [lid, col], mask=m)
                plsc.store_scatter(rows_v, [lid, col], x, mask=m)
            return carry

        lax.fori_loop(j * (CROWS // L), (j + 1) * (CROWS // L), patch, 0)
        ws.append(pltpu.async_copy(rows_v.at[pl.ds(j * CROWS, CROWS)],
                                   out_h.at[pl.ds(base + j * CROWS, CROWS)],
                                   sw))
    for d in ws:
        d.wait()


@jax.jit
def kernel(mem, val, idx):
    scratch = ([pltpu.VMEM((BPW,), jnp.int32) for _ in range(2)]
               + [pltpu.VMEM((CROWS,), jnp.int32) for _ in range(2 * NCHUNK)]
               + [pltpu.VMEM((BPW, DIM), jnp.float32) for _ in range(2)]
               + [pltpu.SemaphoreType.DMA for _ in range(3)])
    return pl.kernel(
        _body,
        out_type=jax.ShapeDtypeStruct((B, DIM), jnp.float32),
        scratch_types=scratch,
        mesh=plsc.VectorSubcoreMesh(core_axis_name="c", subcore_axis_name="s"),
        compiler_params=pltpu.CompilerParams(use_tc_tiling_on_sc=False,
                                             needs_layout_passes=False),
    )(mem, val, idx)
